# baseline (speedup 1.0000x reference)
import jax
import jax.numpy as jnp
from jax import lax
from jax.experimental import pallas as pl
from jax.experimental.pallas import tpu as pltpu
from jax.experimental.pallas import tpu_sc as plsc

_N = 10000
_E = 320000
_DIN = 128
_DH = 64
_W1 = 80
_NT = 16
_ROWS = 632
_N_PAD = _NT * _ROWS
_CH = 128
_TCH = 160
_E_PAD = _NT * _TCH * _CH
_NCH = _E_PAD // _CH
_ST = 32
_NSTAGE = _TCH // _ST

_BLK = 2000
_GRID = _N // _BLK

_REL = ('mc', 'dc', 'cm', 'dm', 'md', 'cd')
_Y_MAP = ((0, 1), (1, 2), (0, 0), (0, 2), (1, 1), (1, 0))


def _make_seg_kernel(width, nbuf):
    mesh = plsc.VectorSubcoreMesh(core_axis_name="c", subcore_axis_name="s")
    out_type = jax.ShapeDtypeStruct((6, _N_PAD, width), jnp.float32)
    scratch_types = [
        pltpu.VMEM((_ST, _CH), jnp.int32),
        pltpu.VMEM((_ST, _CH), jnp.int32),
        pltpu.VMEM((nbuf, _CH, width), jnp.float32),
        pltpu.VMEM_SHARED((_N_PAD, width), jnp.float32),
        pltpu.VMEM_SHARED((_N_PAD, width), jnp.float32),
    ] + [pltpu.SemaphoreType.DMA] * nbuf

    def body(ya, yb, *refs):
        srcs = refs[0:6]
        dsts = refs[6:12]
        out = refs[12]
        (src_v, dst_v, rows_v, acc, tbl) = refs[13:18]
        sems = refs[18:]

        cid = lax.axis_index("c")
        sid = lax.axis_index("s")
        row0 = sid * _ROWS
        ch0 = sid * _TCH
        chunks = []
        r = 0
        while r < _ROWS:
            chunks.append((r, min(_CH, _ROWS - r)))
            r += _CH

        z16 = jnp.zeros((16,), jnp.float32)

        def _fill_zero():
            def _zr(i, c):
                for k in range(width // 16):
                    rows_v[0, i, pl.ds(k * 16, 16)] = z16
                return c
            lax.fori_loop(0, _CH, _zr, 0)

        def run_rel(y_ref, s_ref, d_ref, so_ref):
            for (r0, w) in chunks:
                pltpu.sync_copy(y_ref.at[pl.ds(row0 + r0, w)],
                                rows_v.at[0, pl.ds(0, w)])
                pltpu.sync_copy(rows_v.at[0, pl.ds(0, w)],
                                tbl.at[pl.ds(row0 + r0, w)])
            _fill_zero()
            for (r0, w) in chunks:
                pltpu.sync_copy(rows_v.at[0, pl.ds(0, w)],
                                acc.at[pl.ds(row0 + r0, w)])
            plsc.subcore_barrier()

            def _wait_scatter(j, b):
                pltpu.make_async_copy(tbl.at[src_v.at[j]], rows_v.at[b],
                                      sems[b]).wait()
                pltpu.sync_copy(rows_v.at[b], acc.at[dst_v.at[j]], add=True)

            def _stage(s, c):
                pltpu.sync_copy(s_ref.at[pl.ds(ch0 + s * _ST, _ST)], src_v)
                pltpu.sync_copy(d_ref.at[pl.ds(ch0 + s * _ST, _ST)], dst_v)
                for b in range(nbuf):
                    pltpu.async_copy(tbl.at[src_v.at[b]], rows_v.at[b],
                                     sems[b])

                def _grp(g, c2):
                    for b in range(nbuf):
                        j = g * nbuf + b
                        _wait_scatter(j, b)
                        pltpu.async_copy(tbl.at[src_v.at[j + nbuf]],
                                         rows_v.at[b], sems[b])
                    return c2
                lax.fori_loop(0, _ST // nbuf - 1, _grp, 0)
                for b in range(nbuf):
                    _wait_scatter(_ST - nbuf + b, b)
                return c
            lax.fori_loop(0, _NSTAGE, _stage, 0)
            plsc.subcore_barrier()

            for (r0, w) in chunks:
                pltpu.sync_copy(acc.at[pl.ds(row0 + r0, w)],
                                rows_v.at[0, pl.ds(0, w)])
                pltpu.sync_copy(rows_v.at[0, pl.ds(0, w)],
                                so_ref.at[pl.ds(row0 + r0, w)])

        ytab = (ya, yb)

        @pl.when(cid == 0)
        def _():
            for r in (0, 1, 2):
                t, p = _Y_MAP[r]
                run_rel(ytab[t].at[p], srcs[r], dsts[r], out.at[r])

        @pl.when(cid == 1)
        def _():
            for r in (3, 4, 5):
                t, p = _Y_MAP[r]
                run_rel(ytab[t].at[p], srcs[r], dsts[r], out.at[r])

    return pl.kernel(
        body, out_type=out_type, mesh=mesh, scratch_types=scratch_types,
        compiler_params=pltpu.CompilerParams(use_tc_tiling_on_sc=False))


_seg_l1 = _make_seg_kernel(_W1, 2)
_seg_l2 = _make_seg_kernel(_DH, 4)


def _pre_body(x_ref, w1_ref, w2_ref, o1_ref, o2_ref):
    x = x_ref[0]
    ones = jnp.ones((_BLK, _W1 - _DH), jnp.float32)
    o1_ref[0, :, 0:_DH] = jnp.dot(x, w1_ref[0],
                                  preferred_element_type=jnp.float32)
    o1_ref[0, :, _DH:_W1] = ones
    o2_ref[0, :, 0:_DH] = jnp.dot(x, w2_ref[0],
                                  preferred_element_type=jnp.float32)
    o2_ref[0, :, _DH:_W1] = ones


def _pretransform(x_stack, w1_stack, w2_stack):
    return pl.pallas_call(
        _pre_body,
        grid=(3, _GRID),
        in_specs=[
            pl.BlockSpec((1, _BLK, _DIN), lambda t, i: (t, i, 0)),
            pl.BlockSpec((1, _DIN, _DH), lambda t, i: (t, 0, 0)),
            pl.BlockSpec((1, _DIN, _DH), lambda t, i: (t, 0, 0)),
        ],
        out_specs=[
            pl.BlockSpec((1, _BLK, _W1), lambda t, i: (t, i, 0)),
            pl.BlockSpec((1, _BLK, _W1), lambda t, i: (t, i, 0)),
        ],
        out_shape=[jax.ShapeDtypeStruct((3, _N_PAD, _W1), jnp.float32)] * 2,
    )(x_stack, w1_stack, w2_stack)


def _combine1_body(sa_ref, sb_ref, x_ref, wr_ref, b_ref,
                   wla_ref, wlb_ref, h_ref, ya_ref, yb_ref):
    sa = sa_ref[0]
    sb = sb_ref[0]
    ca = jnp.maximum(sa[:, _DH:_DH + 1], 1.0)
    cb = jnp.maximum(sb[:, _DH:_DH + 1], 1.0)
    agg = 0.5 * (sa[:, 0:_DH] / ca + sb[:, 0:_DH] / cb)
    h = agg + jnp.dot(x_ref[0], wr_ref[0],
                      preferred_element_type=jnp.float32) + b_ref[0]
    h = jnp.maximum(h, 0.0)
    h_ref[0] = h
    ya_ref[0] = jnp.dot(h, wla_ref[0], preferred_element_type=jnp.float32)
    yb_ref[0] = jnp.dot(h, wlb_ref[0], preferred_element_type=jnp.float32)


def _combine1(s1, x_stack, wr, b, wla, wlb):
    return pl.pallas_call(
        _combine1_body,
        grid=(3, _GRID),
        in_specs=[
            pl.BlockSpec((1, _BLK, _W1), lambda t, i: (2 * t, i, 0)),
            pl.BlockSpec((1, _BLK, _W1), lambda t, i: (2 * t + 1, i, 0)),
            pl.BlockSpec((1, _BLK, _DIN), lambda t, i: (t, i, 0)),
            pl.BlockSpec((1, _DIN, _DH), lambda t, i: (t, 0, 0)),
            pl.BlockSpec((1, 1, _DH), lambda t, i: (t, 0, 0)),
            pl.BlockSpec((1, _DH, _DH), lambda t, i: (t, 0, 0)),
            pl.BlockSpec((1, _DH, _DH), lambda t, i: (t, 0, 0)),
        ],
        out_specs=[
            pl.BlockSpec((1, _BLK, _DH), lambda t, i: (t, i, 0)),
            pl.BlockSpec((1, _BLK, _DH), lambda t, i: (t, i, 0)),
            pl.BlockSpec((1, _BLK, _DH), lambda t, i: (t, i, 0)),
        ],
        out_shape=[jax.ShapeDtypeStruct((3, _N, _DH), jnp.float32),
                   jax.ShapeDtypeStruct((3, _N_PAD, _DH), jnp.float32),
                   jax.ShapeDtypeStruct((3, _N_PAD, _DH), jnp.float32)],
    )(s1, s1, x_stack, wr, b, wla, wlb)


def _combine2_body(sa_ref, s1a_ref, sb_ref, s1b_ref, h_ref, x_ref, wr_ref,
                   wres_ref, b_ref, o_ref):
    ca = jnp.maximum(s1a_ref[0][:, _DH:_DH + 1], 1.0)
    cb = jnp.maximum(s1b_ref[0][:, _DH:_DH + 1], 1.0)
    agg = 0.5 * (sa_ref[0] / ca + sb_ref[0] / cb)
    o = (agg
         + jnp.dot(h_ref[0], wr_ref[0], preferred_element_type=jnp.float32)
         + jnp.dot(x_ref[0], wres_ref[0], preferred_element_type=jnp.float32)
         + b_ref[0])
    n = jnp.sqrt(jnp.sum(o * o, axis=1, keepdims=True))
    o_ref[0] = o / jnp.maximum(n, 1e-12)


def _combine2(s2, s1, h_stack, x_stack, wr, wres, b):
    return pl.pallas_call(
        _combine2_body,
        grid=(3, _GRID),
        in_specs=[
            pl.BlockSpec((1, _BLK, _DH), lambda t, i: (2 * t, i, 0)),
            pl.BlockSpec((1, _BLK, _W1), lambda t, i: (2 * t, i, 0)),
            pl.BlockSpec((1, _BLK, _DH), lambda t, i: (2 * t + 1, i, 0)),
            pl.BlockSpec((1, _BLK, _W1), lambda t, i: (2 * t + 1, i, 0)),
            pl.BlockSpec((1, _BLK, _DH), lambda t, i: (t, i, 0)),
            pl.BlockSpec((1, _BLK, _DIN), lambda t, i: (t, i, 0)),
            pl.BlockSpec((1, _DH, _DH), lambda t, i: (t, 0, 0)),
            pl.BlockSpec((1, _DIN, _DH), lambda t, i: (t, 0, 0)),
            pl.BlockSpec((1, 1, _DH), lambda t, i: (t, 0, 0)),
        ],
        out_specs=pl.BlockSpec((1, _BLK, _DH), lambda t, i: (t, i, 0)),
        out_shape=jax.ShapeDtypeStruct((3, _N, _DH), jnp.float32),
    )(s2, s1, s2, s1, h_stack, x_stack, wr, wres, b)


def _prep_idx(src, dst):
    src = src.astype(jnp.int32)
    dst = dst.astype(jnp.int32)
    pad = _E_PAD - _E
    src_p = jnp.concatenate([src, jnp.zeros((pad,), jnp.int32)])
    dst_p = jnp.concatenate([dst, jnp.full((pad,), _N, jnp.int32)])
    return src_p.reshape(_NCH, _CH), dst_p.reshape(_NCH, _CH)


_TY = ('c', 'm', 'd')
_SRC_OF = {'c': ('cm', 'cd'), 'm': ('mc', 'md'), 'd': ('dm', 'dc')}
_DST_OF = {'c': ('mc', 'dc'), 'm': ('cm', 'dm'), 'd': ('md', 'cd')}


def kernel(x_c, x_m, x_d, e_cm, e_md, e_cd, params):
    P1, P2, Pr = params['l1'], params['l2'], params['res']

    edge = {
        'mc': (e_cm[1], e_cm[0]), 'cm': (e_cm[0], e_cm[1]),
        'dm': (e_md[1], e_md[0]), 'md': (e_md[0], e_md[1]),
        'dc': (e_cd[1], e_cd[0]), 'cd': (e_cd[0], e_cd[1]),
    }
    idx = {r: _prep_idx(*edge[r]) for r in _REL}
    idx_args = ([idx[r][0] for r in _REL] + [idx[r][1] for r in _REL])

    x_stack = jnp.stack([x_c, x_m, x_d])

    w1a = jnp.stack([P1[_SRC_OF[t][0]]['Wl'] for t in _TY])
    w1b = jnp.stack([P1[_SRC_OF[t][1]]['Wl'] for t in _TY])
    ya, yb = _pretransform(x_stack, w1a, w1b)

    s1 = _seg_l1(ya, yb, *idx_args)

    wr1 = jnp.stack([0.5 * (P1[_DST_OF[t][0]]['Wr'] + P1[_DST_OF[t][1]]['Wr'])
                     for t in _TY])
    bb1 = jnp.stack([(0.5 * (P1[_DST_OF[t][0]]['bl']
                             + P1[_DST_OF[t][1]]['bl'])).reshape(1, _DH)
                     for t in _TY])
    w2a = jnp.stack([P2[_SRC_OF[t][0]]['Wl'] for t in _TY])
    w2b = jnp.stack([P2[_SRC_OF[t][1]]['Wl'] for t in _TY])
    h_stack, y2a, y2b = _combine1(s1, x_stack, wr1, bb1, w2a, w2b)

    s2 = _seg_l2(y2a, y2b, *idx_args)

    wr2 = jnp.stack([0.5 * (P2[_DST_OF[t][0]]['Wr'] + P2[_DST_OF[t][1]]['Wr'])
                     for t in _TY])
    bb2 = jnp.stack([(0.5 * (P2[_DST_OF[t][0]]['bl'] + P2[_DST_OF[t][1]]['bl'])
                      + Pr[t]['b']).reshape(1, _DH) for t in _TY])
    wres = jnp.stack([Pr[t]['W'] for t in _TY])
    o = _combine2(s2, s1, h_stack, x_stack, wr2, wres, bb2)

    return o[0], o[1], o[2]
